# baseline (speedup 1.0000x reference)
import functools

import jax
import jax.numpy as jnp
from jax import lax
from jax.experimental import pallas as pl
from jax.experimental.pallas import tpu as pltpu
from jax.experimental.pallas import tpu_sc as plsc

_NC = 2
_NS = 16
_LANE = 128


def _emb_body(x_ref, wf_ref, bf_ref, o_ref):
    o_ref[...] = (
        jnp.dot(x_ref[...], wf_ref[...], preferred_element_type=jnp.float32)
        + bf_ref[...]
    )


def _prep_body(ei_ref, ety_ref, etm_ref, m_ref, lam_ref,
               gia_ref, gib_ref, dst_ref, w_ref, *, n):
    src = ei_ref[0]
    dst = ei_ref[1]
    ety = ety_ref[...]
    erows = src.shape[0]
    rows_pad = gia_ref.shape[0]
    zi = jnp.zeros((rows_pad - erows, src.shape[1]), jnp.int32)
    gia_ref[...] = jnp.concatenate([ety * n + src, zi])
    gib_ref[...] = jnp.concatenate([ety * n + dst, zi])
    dst_ref[...] = jnp.concatenate([dst, zi])
    logit = jnp.dot(etm_ref[...], m_ref[...], preferred_element_type=jnp.float32)
    w = lam_ref[0, 0] * jnp.exp(-logit)
    zf = jnp.zeros((w_ref.shape[0] - w.shape[0], w.shape[1]), jnp.float32)
    w_ref[...] = jnp.concatenate([w, zf])


def _tables_body(h_ref, wt_ref, wb_ref, br_ref, ta_ref, tb_ref):
    h = h_ref[...]
    ta_ref[0] = jnp.dot(h, wt_ref[0], preferred_element_type=jnp.float32)
    tb_ref[0] = (
        jnp.dot(h, wb_ref[0], preferred_element_type=jnp.float32) + br_ref[0]
    )


def _tables_sum_body(p_ref, wt_ref, wb_ref, br_ref, ta_ref, tb_ref, h_ref):
    h = p_ref[0] + p_ref[1]
    h_ref[...] = h
    ta_ref[0] = jnp.dot(h, wt_ref[0], preferred_element_type=jnp.float32)
    tb_ref[0] = (
        jnp.dot(h, wb_ref[0], preferred_element_type=jnp.float32) + br_ref[0]
    )


def _final_body(h0_ref, h1_ref, p2_ref, w0_ref, b0_ref, w1_ref, b1_ref,
                w2_ref, b2_ref, o_ref):
    def lrelu(z):
        return jnp.where(z > 0, z, 0.01 * z)

    h2 = p2_ref[0] + p2_ref[1]
    acc = lrelu(jnp.dot(h0_ref[...], w0_ref[...],
                        preferred_element_type=jnp.float32) + b0_ref[...])
    acc += lrelu(jnp.dot(h1_ref[...], w1_ref[...],
                         preferred_element_type=jnp.float32) + b1_ref[...])
    acc += lrelu(jnp.dot(h2, w2_ref[...],
                         preferred_element_type=jnp.float32) + b2_ref[...])
    o_ref[...] = acc


def _sc_layer(ta, tb, gia, gib, dstr, wr, n_pad, H):
    rows_total = gia.shape[0]
    nw = _NC * _NS
    rw = rows_total // nw
    ch_rows = 4
    n_chunks = rw // ch_rows
    ch = ch_rows * _LANE
    nrows = n_pad // _NS

    mesh = plsc.VectorSubcoreMesh(core_axis_name="c", subcore_axis_name="s")

    @functools.partial(
        pl.kernel,
        out_type=jax.ShapeDtypeStruct((_NC, n_pad, H), jnp.float32),
        mesh=mesh,
        scratch_types=[
            pltpu.VMEM((rw, _LANE), jnp.int32),
            pltpu.VMEM((rw, _LANE), jnp.int32),
            pltpu.VMEM((rw, _LANE), jnp.int32),
            pltpu.VMEM((rw * _LANE // H, H), jnp.float32),
            pltpu.VMEM((ch, H), jnp.float32),
            pltpu.VMEM((ch, H), jnp.float32),
            pltpu.VMEM((ch, H), jnp.float32),
            pltpu.VMEM((ch, H), jnp.float32),
            pltpu.VMEM_SHARED((n_pad, H), jnp.float32),
            pltpu.SemaphoreType.DMA,
            pltpu.SemaphoreType.DMA,
            pltpu.SemaphoreType.DMA,
            pltpu.SemaphoreType.DMA,
        ],
        compiler_params=pltpu.CompilerParams(use_tc_tiling_on_sc=False),
    )
    def sck(gia_h, gib_h, dst_h, w_h, ta_h, tb_h, out_h,
            idxa, idxb, dstv, wv, bufa0, bufa1, bufb0, bufb1, accum,
            sg0, sg1, ss0, ss1):
        cid = lax.axis_index("c")
        sid = lax.axis_index("s")
        wid = cid * _NS + sid
        base = sid * nrows
        bufa = (bufa0, bufa1)
        bufb = (bufb0, bufb1)
        sg = (sg0, sg1)
        ss = (ss0, ss1)

        def zero_row(i, carry):
            z = jnp.zeros((16,), jnp.float32)
            bufa0[i, pl.ds(0, 16)] = z
            bufa0[i, pl.ds(16, 16)] = z
            return carry

        lax.fori_loop(0, ch, zero_row, 0, unroll=4)
        pltpu.sync_copy(bufa0.at[pl.ds(0, ch)], accum.at[pl.ds(base, ch)])
        rem = nrows - ch
        pltpu.sync_copy(bufa0.at[pl.ds(0, rem)],
                        accum.at[pl.ds(base + ch, rem)])
        plsc.subcore_barrier()

        erow0 = wid * rw
        pltpu.sync_copy(gia_h.at[pl.ds(erow0, rw)], idxa)
        pltpu.sync_copy(gib_h.at[pl.ds(erow0, rw)], idxb)
        pltpu.sync_copy(dst_h.at[pl.ds(erow0, rw)], dstv)
        rw4 = rw * _LANE // H
        pltpu.sync_copy(w_h.at[pl.ds(wid * rw4, rw4)], wv)

        def fire_gathers(ci, s):
            r0 = ci * ch_rows
            cps = []
            for j in range(ch_rows):
                cps.append(pltpu.async_copy(
                    ta_h.at[idxa.at[r0 + j]],
                    bufa[s].at[pl.ds(j * _LANE, _LANE)], sg[s]))
                cps.append(pltpu.async_copy(
                    tb_h.at[idxb.at[r0 + j]],
                    bufb[s].at[pl.ds(j * _LANE, _LANE)], sg[s]))
            return cps

        def compute_chunk(ci, s):
            ba = bufa[s]
            bb = bufb[s]
            r0w = ci * (ch // H)

            def group_body(g, carry2):
                gr = r0w + lax.shift_right_logical(g, 1)
                gl = lax.bitwise_and(g, 1) * 16
                w16 = wv[gr, pl.ds(gl, 16)]
                e0 = g * 16
                for k in range(16):
                    wvec = lax.gather(
                        w16, jnp.full((16, 1), k, jnp.int32),
                        lax.GatherDimensionNumbers(
                            offset_dims=(), collapsed_slice_dims=(0,),
                            start_index_map=(0,)),
                        slice_sizes=(1,),
                        mode=lax.GatherScatterMode.PROMISE_IN_BOUNDS)
                    e = e0 + k
                    lo = pl.ds(0, 16)
                    hi = pl.ds(16, 16)
                    ba[e, lo] = (ba[e, lo] + bb[e, lo]) * wvec
                    ba[e, hi] = (ba[e, hi] + bb[e, hi]) * wvec
                return carry2

            lax.fori_loop(0, ch // 16, group_body, 0)

        def fire_scatters(ci, s):
            r0 = ci * ch_rows
            cps = []
            for j in range(ch_rows):
                cps.append(pltpu.async_copy(
                    bufa[s].at[pl.ds(j * _LANE, _LANE)],
                    accum.at[dstv.at[r0 + j]], ss[s], add=True))
            return cps

        pend_scatter = [None, None]
        pend_gather = [None, None]
        pend_gather[0] = fire_gathers(0, 0)
        for ci in range(n_chunks):
            s = ci % 2
            o = 1 - s
            if ci + 1 < n_chunks:
                if pend_scatter[o] is not None:
                    for cp in pend_scatter[o]:
                        cp.wait()
                    pend_scatter[o] = None
                pend_gather[o] = fire_gathers(ci + 1, o)
            for cp in pend_gather[s]:
                cp.wait()
            compute_chunk(ci, s)
            pend_scatter[s] = fire_scatters(ci, s)
        for s in range(2):
            if pend_scatter[s] is not None:
                for cp in pend_scatter[s]:
                    cp.wait()

        plsc.subcore_barrier()
        pltpu.sync_copy(accum.at[pl.ds(base, ch)], bufa0.at[pl.ds(0, ch)])
        pltpu.sync_copy(bufa0.at[pl.ds(0, ch)],
                        out_h.at[cid, pl.ds(base, ch)])
        pltpu.sync_copy(accum.at[pl.ds(base + ch, rem)],
                        bufa1.at[pl.ds(0, rem)])
        pltpu.sync_copy(bufa1.at[pl.ds(0, rem)],
                        out_h.at[cid, pl.ds(base + ch, rem)])

    return sck(gia, gib, dstr, wr, ta, tb)


def kernel(x, edge_index, edge_type, edge_time, lambda_sym, beta, Wf, bf,
           Wr1, br1, Wr2, br2, W0, b0, W1, b1, W2, b2):
    N, in_dim = x.shape
    H = Wf.shape[1]
    R = Wr1.shape[0]
    E = edge_index.shape[1]
    out_dim = W0.shape[1]
    f32 = jnp.float32

    nb = 10
    bn = N // nb

    h0 = pl.pallas_call(
        _emb_body,
        grid=(nb,),
        in_specs=[
            pl.BlockSpec((bn, in_dim), lambda i: (i, 0)),
            pl.BlockSpec((in_dim, H), lambda i: (0, 0)),
            pl.BlockSpec((1, H), lambda i: (0, 0)),
        ],
        out_specs=pl.BlockSpec((bn, H), lambda i: (i, 0)),
        out_shape=jax.ShapeDtypeStruct((N, H), f32),
    )(x, Wf, bf.reshape(1, H))

    group = _NC * _NS * _LANE * 8
    e_pad = ((E + group - 1) // group) * group
    n_pad = ((N + 8 * _NS - 1) // (8 * _NS)) * (8 * _NS)
    tdim = edge_time.shape[1]
    erows = E // _LANE
    rows_pad = e_pad // _LANE
    wcols = H
    wrows = E * tdim // (tdim * wcols)
    ei3 = edge_index.astype(jnp.int32).reshape(2, erows, _LANE)
    ety2 = edge_type.astype(jnp.int32).reshape(erows, _LANE)
    etm = edge_time.reshape(E // wcols, tdim * wcols)
    mmat = jnp.kron(jnp.eye(wcols, dtype=f32), beta)
    gia_p, gib_p, dst_p, w_p = pl.pallas_call(
        functools.partial(_prep_body, n=N),
        grid=(1,),
        in_specs=[
            pl.BlockSpec((2, erows, _LANE), lambda i: (0, 0, 0)),
            pl.BlockSpec((erows, _LANE), lambda i: (0, 0)),
            pl.BlockSpec((E // wcols, tdim * wcols), lambda i: (0, 0)),
            pl.BlockSpec((tdim * wcols, wcols), lambda i: (0, 0)),
            pl.BlockSpec((1, 1), lambda i: (0, 0)),
        ],
        out_specs=[
            pl.BlockSpec((rows_pad, _LANE), lambda i: (0, 0)),
            pl.BlockSpec((rows_pad, _LANE), lambda i: (0, 0)),
            pl.BlockSpec((rows_pad, _LANE), lambda i: (0, 0)),
            pl.BlockSpec((e_pad // wcols, wcols), lambda i: (0, 0)),
        ],
        out_shape=[
            jax.ShapeDtypeStruct((rows_pad, _LANE), jnp.int32),
            jax.ShapeDtypeStruct((rows_pad, _LANE), jnp.int32),
            jax.ShapeDtypeStruct((rows_pad, _LANE), jnp.int32),
            jax.ShapeDtypeStruct((e_pad // wcols, wcols), f32),
        ],
    )(ei3, ety2, etm, mmat, lambda_sym)

    tbl_specs = dict(
        grid=(nb, R),
        out_shape=[
            jax.ShapeDtypeStruct((R, N, H), f32),
            jax.ShapeDtypeStruct((R, N, H), f32),
        ],
    )
    wt_spec = pl.BlockSpec((1, H, H), lambda i, r: (r, 0, 0))
    br_spec = pl.BlockSpec((1, 1, H), lambda i, r: (r, 0, 0))
    t_out = [
        pl.BlockSpec((1, bn, H), lambda i, r: (r, i, 0)),
        pl.BlockSpec((1, bn, H), lambda i, r: (r, i, 0)),
    ]

    wt1 = Wr1[:, :H, :]
    wb1 = Wr1[:, H:, :]
    ta1, tb1 = pl.pallas_call(
        _tables_body,
        in_specs=[
            pl.BlockSpec((bn, H), lambda i, r: (i, 0)),
            wt_spec, wt_spec, br_spec,
        ],
        out_specs=t_out,
        **tbl_specs,
    )(h0, wt1, wb1, br1.reshape(R, 1, H))

    p1 = _sc_layer(ta1.reshape(R * N, H), tb1.reshape(R * N, H),
                   gia_p, gib_p, dst_p, w_p, n_pad, H)

    wt2 = Wr2[:, :H, :]
    wb2 = Wr2[:, H:, :]
    ta2, tb2, h1 = pl.pallas_call(
        _tables_sum_body,
        grid=(nb, R),
        in_specs=[
            pl.BlockSpec((_NC, bn, H), lambda i, r: (0, i, 0)),
            wt_spec, wt_spec, br_spec,
        ],
        out_specs=t_out + [pl.BlockSpec((bn, H), lambda i, r: (i, 0))],
        out_shape=[
            jax.ShapeDtypeStruct((R, N, H), f32),
            jax.ShapeDtypeStruct((R, N, H), f32),
            jax.ShapeDtypeStruct((N, H), f32),
        ],
    )(p1, wt2, wb2, br2.reshape(R, 1, H))

    p2 = _sc_layer(ta2.reshape(R * N, H), tb2.reshape(R * N, H),
                   gia_p, gib_p, dst_p, w_p, n_pad, H)

    out = pl.pallas_call(
        _final_body,
        grid=(nb,),
        in_specs=[
            pl.BlockSpec((bn, H), lambda i: (i, 0)),
            pl.BlockSpec((bn, H), lambda i: (i, 0)),
            pl.BlockSpec((_NC, bn, H), lambda i: (0, i, 0)),
            pl.BlockSpec((H, out_dim), lambda i: (0, 0)),
            pl.BlockSpec((1, out_dim), lambda i: (0, 0)),
            pl.BlockSpec((H, out_dim), lambda i: (0, 0)),
            pl.BlockSpec((1, out_dim), lambda i: (0, 0)),
            pl.BlockSpec((H, out_dim), lambda i: (0, 0)),
            pl.BlockSpec((1, out_dim), lambda i: (0, 0)),
        ],
        out_specs=pl.BlockSpec((bn, out_dim), lambda i: (i, 0)),
        out_shape=jax.ShapeDtypeStruct((N, out_dim), f32),
    )(h0, h1, p2, W0, b0.reshape(1, out_dim), W1, b1.reshape(1, out_dim),
      W2, b2.reshape(1, out_dim))

    return out


# speedup vs baseline: 1.2370x; 1.2370x over previous
import functools

import jax
import jax.numpy as jnp
from jax import lax
from jax.experimental import pallas as pl
from jax.experimental.pallas import tpu as pltpu
from jax.experimental.pallas import tpu_sc as plsc

_NC = 2
_NS = 16
_LANE = 128


def _emb_body(x_ref, wf_ref, bf_ref, o_ref):
    o_ref[...] = (
        jnp.dot(x_ref[...], wf_ref[...], preferred_element_type=jnp.float32)
        + bf_ref[...]
    )


def _prep_body(ei_ref, ety_ref, lg_ref, lam_ref,
               gia_ref, gib_ref, dst_ref, w_ref, *, n):
    src = ei_ref[0]
    dst = ei_ref[1]
    ety = ety_ref[...]
    erows = src.shape[0]
    rows_pad = gia_ref.shape[0]
    zi = jnp.zeros((rows_pad - erows, src.shape[1]), jnp.int32)
    gia_ref[...] = jnp.concatenate([ety * n + src, zi])
    gib_ref[...] = jnp.concatenate([ety * n + dst, zi])
    dst_ref[...] = jnp.concatenate([dst, zi])
    w = lam_ref[0, 0] * jnp.exp(-lg_ref[...])
    zf = jnp.zeros((w_ref.shape[0] - w.shape[0], w.shape[1]), jnp.float32)
    w_ref[...] = jnp.concatenate([w, zf])


def _tables_body(h_ref, wt_ref, wb_ref, br_ref, ta_ref, tb_ref):
    h = h_ref[...]
    ta_ref[...] = jnp.dot(h, wt_ref[0], preferred_element_type=jnp.float32)
    tb_ref[...] = (
        jnp.dot(h, wb_ref[0], preferred_element_type=jnp.float32) + br_ref[0]
    )


def _tables_sum_body(p_ref, wt_ref, wb_ref, br_ref, ta_ref, tb_ref, h_ref):
    n = h_ref.shape[0]
    h = (p_ref[0] + p_ref[1])[:n]
    h_ref[...] = h
    ta_ref[...] = jnp.dot(h, wt_ref[0], preferred_element_type=jnp.float32)
    tb_ref[...] = (
        jnp.dot(h, wb_ref[0], preferred_element_type=jnp.float32) + br_ref[0]
    )


def _final_body(h0_ref, h1_ref, p2_ref, w0_ref, b0_ref, w1_ref, b1_ref,
                w2_ref, b2_ref, o_ref):
    def lrelu(z):
        return jnp.where(z > 0, z, 0.01 * z)

    h2 = p2_ref[0] + p2_ref[1]
    acc = lrelu(jnp.dot(h0_ref[...], w0_ref[...],
                        preferred_element_type=jnp.float32) + b0_ref[...])
    acc += lrelu(jnp.dot(h1_ref[...], w1_ref[...],
                         preferred_element_type=jnp.float32) + b1_ref[...])
    acc += lrelu(jnp.dot(h2, w2_ref[...],
                         preferred_element_type=jnp.float32) + b2_ref[...])
    o_ref[...] = acc


def _sc_layer(ta, tb, gia, gib, dstr, wr, n_pad, H):
    rows_total = gia.shape[0]
    nw = _NC * _NS
    rw = rows_total // nw
    ch_rows = 4
    n_chunks = rw // ch_rows
    ch = ch_rows * _LANE
    nrows = n_pad // _NS

    mesh = plsc.VectorSubcoreMesh(core_axis_name="c", subcore_axis_name="s")

    @functools.partial(
        pl.kernel,
        out_type=jax.ShapeDtypeStruct((_NC, n_pad, H), jnp.float32),
        mesh=mesh,
        scratch_types=[
            pltpu.VMEM((rw, _LANE), jnp.int32),
            pltpu.VMEM((rw, _LANE), jnp.int32),
            pltpu.VMEM((rw, _LANE), jnp.int32),
            pltpu.VMEM((rw * _LANE // H, H), jnp.float32),
            pltpu.VMEM((ch, H), jnp.float32),
            pltpu.VMEM((ch, H), jnp.float32),
            pltpu.VMEM((ch, H), jnp.float32),
            pltpu.VMEM((ch, H), jnp.float32),
            pltpu.VMEM_SHARED((n_pad, H), jnp.float32),
            pltpu.SemaphoreType.DMA,
            pltpu.SemaphoreType.DMA,
            pltpu.SemaphoreType.DMA,
            pltpu.SemaphoreType.DMA,
        ],
        compiler_params=pltpu.CompilerParams(use_tc_tiling_on_sc=False),
    )
    def sck(gia_h, gib_h, dst_h, w_h, ta_h, tb_h, out_h,
            idxa, idxb, dstv, wv, bufa0, bufa1, bufb0, bufb1, accum,
            sg0, sg1, ss0, ss1):
        cid = lax.axis_index("c")
        sid = lax.axis_index("s")
        wid = cid * _NS + sid
        base = sid * nrows
        bufa = (bufa0, bufa1)
        bufb = (bufb0, bufb1)
        sg = (sg0, sg1)
        ss = (ss0, ss1)

        def zero_row(i, carry):
            z = jnp.zeros((16,), jnp.float32)
            bufa0[i, pl.ds(0, 16)] = z
            bufa0[i, pl.ds(16, 16)] = z
            return carry

        lax.fori_loop(0, ch, zero_row, 0, unroll=4)
        pltpu.sync_copy(bufa0.at[pl.ds(0, ch)], accum.at[pl.ds(base, ch)])
        rem = nrows - ch
        pltpu.sync_copy(bufa0.at[pl.ds(0, rem)],
                        accum.at[pl.ds(base + ch, rem)])
        plsc.subcore_barrier()

        erow0 = wid * rw
        pltpu.sync_copy(gia_h.at[pl.ds(erow0, rw)], idxa)
        pltpu.sync_copy(gib_h.at[pl.ds(erow0, rw)], idxb)
        pltpu.sync_copy(dst_h.at[pl.ds(erow0, rw)], dstv)
        rw4 = rw * _LANE // H
        pltpu.sync_copy(w_h.at[pl.ds(wid * rw4, rw4)], wv)

        def fire_gathers(ci, s):
            r0 = ci * ch_rows
            cps = []
            for j in range(ch_rows):
                cps.append(pltpu.async_copy(
                    ta_h.at[idxa.at[r0 + j]],
                    bufa[s].at[pl.ds(j * _LANE, _LANE)], sg[s]))
                cps.append(pltpu.async_copy(
                    tb_h.at[idxb.at[r0 + j]],
                    bufb[s].at[pl.ds(j * _LANE, _LANE)], sg[s]))
            return cps

        def compute_chunk(ci, s):
            ba = bufa[s]
            bb = bufb[s]
            r0w = ci * (ch // H)

            def group_body(g, carry2):
                gr = r0w + lax.shift_right_logical(g, 1)
                gl = lax.bitwise_and(g, 1) * 16
                w16 = wv[gr, pl.ds(gl, 16)]
                e0 = g * 16
                for k in range(16):
                    wvec = lax.gather(
                        w16, jnp.full((16, 1), k, jnp.int32),
                        lax.GatherDimensionNumbers(
                            offset_dims=(), collapsed_slice_dims=(0,),
                            start_index_map=(0,)),
                        slice_sizes=(1,),
                        mode=lax.GatherScatterMode.PROMISE_IN_BOUNDS)
                    e = e0 + k
                    lo = pl.ds(0, 16)
                    hi = pl.ds(16, 16)
                    ba[e, lo] = (ba[e, lo] + bb[e, lo]) * wvec
                    ba[e, hi] = (ba[e, hi] + bb[e, hi]) * wvec
                return carry2

            lax.fori_loop(0, ch // 16, group_body, 0)

        def fire_scatters(ci, s):
            r0 = ci * ch_rows
            cps = []
            for j in range(ch_rows):
                cps.append(pltpu.async_copy(
                    bufa[s].at[pl.ds(j * _LANE, _LANE)],
                    accum.at[dstv.at[r0 + j]], ss[s], add=True))
            return cps

        pend_scatter = [None, None]
        pend_gather = [None, None]
        pend_gather[0] = fire_gathers(0, 0)
        for ci in range(n_chunks):
            s = ci % 2
            o = 1 - s
            if ci + 1 < n_chunks:
                if pend_scatter[o] is not None:
                    for cp in pend_scatter[o]:
                        cp.wait()
                    pend_scatter[o] = None
                pend_gather[o] = fire_gathers(ci + 1, o)
            for cp in pend_gather[s]:
                cp.wait()
            compute_chunk(ci, s)
            pend_scatter[s] = fire_scatters(ci, s)
        for s in range(2):
            if pend_scatter[s] is not None:
                for cp in pend_scatter[s]:
                    cp.wait()

        plsc.subcore_barrier()
        pltpu.sync_copy(accum.at[pl.ds(base, ch)], bufa0.at[pl.ds(0, ch)])
        pltpu.sync_copy(bufa0.at[pl.ds(0, ch)],
                        out_h.at[cid, pl.ds(base, ch)])
        pltpu.sync_copy(accum.at[pl.ds(base + ch, rem)],
                        bufa1.at[pl.ds(0, rem)])
        pltpu.sync_copy(bufa1.at[pl.ds(0, rem)],
                        out_h.at[cid, pl.ds(base + ch, rem)])

    return sck(gia, gib, dstr, wr, ta, tb)


def kernel(x, edge_index, edge_type, edge_time, lambda_sym, beta, Wf, bf,
           Wr1, br1, Wr2, br2, W0, b0, W1, b1, W2, b2):
    N, in_dim = x.shape
    H = Wf.shape[1]
    R = Wr1.shape[0]
    E = edge_index.shape[1]
    out_dim = W0.shape[1]
    f32 = jnp.float32

    nb = 10
    bn = N // nb

    h0 = pl.pallas_call(
        _emb_body,
        grid=(nb,),
        in_specs=[
            pl.BlockSpec((bn, in_dim), lambda i: (i, 0)),
            pl.BlockSpec((in_dim, H), lambda i: (0, 0)),
            pl.BlockSpec((1, H), lambda i: (0, 0)),
        ],
        out_specs=pl.BlockSpec((bn, H), lambda i: (i, 0)),
        out_shape=jax.ShapeDtypeStruct((N, H), f32),
    )(x, Wf, bf.reshape(1, H))

    group = _NC * _NS * _LANE * 8
    e_pad = ((E + group - 1) // group) * group
    n_pad = ((N + 8 * _NS - 1) // (8 * _NS)) * (8 * _NS)
    erows = E // _LANE
    rows_pad = e_pad // _LANE
    wcols = H
    ei3 = edge_index.astype(jnp.int32).reshape(2, erows, _LANE)
    ety2 = edge_type.astype(jnp.int32).reshape(erows, _LANE)
    lg = jnp.dot(edge_time, beta).reshape(E // wcols, wcols)
    gia_p, gib_p, dst_p, w_p = pl.pallas_call(
        functools.partial(_prep_body, n=N),
        grid=(1,),
        in_specs=[
            pl.BlockSpec((2, erows, _LANE), lambda i: (0, 0, 0)),
            pl.BlockSpec((erows, _LANE), lambda i: (0, 0)),
            pl.BlockSpec((E // wcols, wcols), lambda i: (0, 0)),
            pl.BlockSpec((1, 1), lambda i: (0, 0)),
        ],
        out_specs=[
            pl.BlockSpec((rows_pad, _LANE), lambda i: (0, 0)),
            pl.BlockSpec((rows_pad, _LANE), lambda i: (0, 0)),
            pl.BlockSpec((rows_pad, _LANE), lambda i: (0, 0)),
            pl.BlockSpec((e_pad // wcols, wcols), lambda i: (0, 0)),
        ],
        out_shape=[
            jax.ShapeDtypeStruct((rows_pad, _LANE), jnp.int32),
            jax.ShapeDtypeStruct((rows_pad, _LANE), jnp.int32),
            jax.ShapeDtypeStruct((rows_pad, _LANE), jnp.int32),
            jax.ShapeDtypeStruct((e_pad // wcols, wcols), f32),
        ],
    )(ei3, ety2, lg, lambda_sym)

    wt_spec = pl.BlockSpec((1, H, H), lambda r: (r, 0, 0))
    br_spec = pl.BlockSpec((1, 1, H), lambda r: (r, 0, 0))
    t_out = [
        pl.BlockSpec((N, H), lambda r: (r, 0)),
        pl.BlockSpec((N, H), lambda r: (r, 0)),
    ]
    t_shape = [
        jax.ShapeDtypeStruct((R * N, H), f32),
        jax.ShapeDtypeStruct((R * N, H), f32),
    ]

    wt1 = Wr1[:, :H, :]
    wb1 = Wr1[:, H:, :]
    ta1, tb1 = pl.pallas_call(
        _tables_body,
        grid=(R,),
        in_specs=[
            pl.BlockSpec((N, H), lambda r: (0, 0)),
            wt_spec, wt_spec, br_spec,
        ],
        out_specs=t_out,
        out_shape=t_shape,
    )(h0, wt1, wb1, br1.reshape(R, 1, H))

    p1 = _sc_layer(ta1, tb1, gia_p, gib_p, dst_p, w_p, n_pad, H)

    wt2 = Wr2[:, :H, :]
    wb2 = Wr2[:, H:, :]
    ta2, tb2, h1 = pl.pallas_call(
        _tables_sum_body,
        grid=(R,),
        in_specs=[
            pl.BlockSpec((_NC, n_pad, H), lambda r: (0, 0, 0)),
            wt_spec, wt_spec, br_spec,
        ],
        out_specs=t_out + [pl.BlockSpec((N, H), lambda r: (0, 0))],
        out_shape=t_shape + [jax.ShapeDtypeStruct((N, H), f32)],
    )(p1, wt2, wb2, br2.reshape(R, 1, H))

    p2 = _sc_layer(ta2, tb2, gia_p, gib_p, dst_p, w_p, n_pad, H)

    out = pl.pallas_call(
        _final_body,
        grid=(nb,),
        in_specs=[
            pl.BlockSpec((bn, H), lambda i: (i, 0)),
            pl.BlockSpec((bn, H), lambda i: (i, 0)),
            pl.BlockSpec((_NC, bn, H), lambda i: (0, i, 0)),
            pl.BlockSpec((H, out_dim), lambda i: (0, 0)),
            pl.BlockSpec((1, out_dim), lambda i: (0, 0)),
            pl.BlockSpec((H, out_dim), lambda i: (0, 0)),
            pl.BlockSpec((1, out_dim), lambda i: (0, 0)),
            pl.BlockSpec((H, out_dim), lambda i: (0, 0)),
            pl.BlockSpec((1, out_dim), lambda i: (0, 0)),
        ],
        out_specs=pl.BlockSpec((bn, out_dim), lambda i: (i, 0)),
        out_shape=jax.ShapeDtypeStruct((N, out_dim), f32),
    )(h0, h1, p2, W0, b0.reshape(1, out_dim), W1, b1.reshape(1, out_dim),
      W2, b2.reshape(1, out_dim))

    return out


# speedup vs baseline: 1.6012x; 1.2944x over previous
import functools

import jax
import jax.numpy as jnp
from jax import lax
from jax.experimental import pallas as pl
from jax.experimental.pallas import tpu as pltpu
from jax.experimental.pallas import tpu_sc as plsc

_NC = 2
_NS = 16
_LANE = 128


def _emb_body(x_ref, wf_ref, bf_ref, o_ref):
    o_ref[...] = (
        jnp.dot(x_ref[...], wf_ref[...], preferred_element_type=jnp.float32)
        + bf_ref[...]
    )


def _prep_body(ei_ref, ety_ref, ett_ref, beta_ref, lam_ref,
               gia_ref, gib_ref, dst_ref, w_ref, *, r):
    src = ei_ref[0]
    dst = ei_ref[1]
    ety = ety_ref[...]
    erows = src.shape[0]
    rows_pad = gia_ref.shape[0]
    zi = jnp.zeros((rows_pad - erows, src.shape[1]), jnp.int32)
    gia_ref[...] = jnp.concatenate([src * r + ety, zi])
    gib_ref[...] = jnp.concatenate([dst * r + ety, zi])
    dst_ref[...] = jnp.concatenate([dst, zi])
    tdim = ett_ref.shape[0]
    logit = ett_ref[0] * beta_ref[0, 0]
    for k in range(1, tdim):
        logit += ett_ref[k] * beta_ref[0, k]
    w = lam_ref[0, 0] * jnp.exp(-logit)
    zf = jnp.zeros((rows_pad - erows, src.shape[1]), jnp.float32)
    w_ref[...] = jnp.concatenate([w, zf])


def _tables_body(h_ref, wt_ref, wb_ref, br_ref, ta_ref, tb_ref):
    h = h_ref[...]
    ta_ref[...] = jnp.dot(h, wt_ref[...], preferred_element_type=jnp.float32)
    tb_ref[...] = (
        jnp.dot(h, wb_ref[...], preferred_element_type=jnp.float32)
        + br_ref[...]
    )


def _tables_sum_body(p_ref, wt_ref, wb_ref, br_ref, ta_ref, tb_ref, h_ref):
    n = h_ref.shape[0]
    h = (p_ref[0] + p_ref[1])[:n]
    h_ref[...] = h
    ta_ref[...] = jnp.dot(h, wt_ref[...], preferred_element_type=jnp.float32)
    tb_ref[...] = (
        jnp.dot(h, wb_ref[...], preferred_element_type=jnp.float32)
        + br_ref[...]
    )


def _final_body(h0_ref, h1_ref, p2_ref, w0_ref, b0_ref, w1_ref, b1_ref,
                w2_ref, b2_ref, o_ref):
    def lrelu(z):
        return jnp.where(z > 0, z, 0.01 * z)

    h2 = p2_ref[0] + p2_ref[1]
    acc = lrelu(jnp.dot(h0_ref[...], w0_ref[...],
                        preferred_element_type=jnp.float32) + b0_ref[...])
    acc += lrelu(jnp.dot(h1_ref[...], w1_ref[...],
                         preferred_element_type=jnp.float32) + b1_ref[...])
    acc += lrelu(jnp.dot(h2, w2_ref[...],
                         preferred_element_type=jnp.float32) + b2_ref[...])
    o_ref[...] = acc


def _sc_layer(ta, tb, gia, gib, dstr, wr, n_pad, H):
    rows_total = gia.shape[0]
    nw = _NC * _NS
    rw = rows_total // nw
    ch_rows = 4
    n_chunks = rw // ch_rows
    ch = ch_rows * _LANE
    nrows = n_pad // _NS

    mesh = plsc.VectorSubcoreMesh(core_axis_name="c", subcore_axis_name="s")

    @functools.partial(
        pl.kernel,
        out_type=jax.ShapeDtypeStruct((_NC, n_pad, H), jnp.float32),
        mesh=mesh,
        scratch_types=[
            pltpu.VMEM((rw, _LANE), jnp.int32),
            pltpu.VMEM((rw, _LANE), jnp.int32),
            pltpu.VMEM((rw, _LANE), jnp.int32),
            pltpu.VMEM((rw, _LANE), jnp.float32),
            pltpu.VMEM((ch, H), jnp.float32),
            pltpu.VMEM((ch, H), jnp.float32),
            pltpu.VMEM((ch, H), jnp.float32),
            pltpu.VMEM((ch, H), jnp.float32),
            pltpu.VMEM_SHARED((n_pad, H), jnp.float32),
            pltpu.SemaphoreType.DMA,
            pltpu.SemaphoreType.DMA,
            pltpu.SemaphoreType.DMA,
            pltpu.SemaphoreType.DMA,
        ],
        compiler_params=pltpu.CompilerParams(use_tc_tiling_on_sc=False),
    )
    def sck(gia_h, gib_h, dst_h, w_h, ta_h, tb_h, out_h,
            idxa, idxb, dstv, wv, bufa0, bufa1, bufb0, bufb1, accum,
            sg0, sg1, ss0, ss1):
        cid = lax.axis_index("c")
        sid = lax.axis_index("s")
        wid = cid * _NS + sid
        base = sid * nrows
        bufa = (bufa0, bufa1)
        bufb = (bufb0, bufb1)
        sg = (sg0, sg1)
        ss = (ss0, ss1)

        def zero_row(i, carry):
            z = jnp.zeros((16,), jnp.float32)
            bufa0[i, pl.ds(0, 16)] = z
            bufa0[i, pl.ds(16, 16)] = z
            return carry

        lax.fori_loop(0, ch, zero_row, 0, unroll=4)
        pltpu.sync_copy(bufa0.at[pl.ds(0, ch)], accum.at[pl.ds(base, ch)])
        rem = nrows - ch
        pltpu.sync_copy(bufa0.at[pl.ds(0, rem)],
                        accum.at[pl.ds(base + ch, rem)])
        plsc.subcore_barrier()

        erow0 = wid * rw
        pltpu.sync_copy(gia_h.at[pl.ds(erow0, rw)], idxa)
        pltpu.sync_copy(gib_h.at[pl.ds(erow0, rw)], idxb)
        pltpu.sync_copy(dst_h.at[pl.ds(erow0, rw)], dstv)
        pltpu.sync_copy(w_h.at[pl.ds(erow0, rw)], wv)

        def fire_gathers(ci, s):
            r0 = ci * ch_rows
            cps = []
            for j in range(ch_rows):
                cps.append(pltpu.async_copy(
                    ta_h.at[idxa.at[r0 + j]],
                    bufa[s].at[pl.ds(j * _LANE, _LANE)], sg[s]))
                cps.append(pltpu.async_copy(
                    tb_h.at[idxb.at[r0 + j]],
                    bufb[s].at[pl.ds(j * _LANE, _LANE)], sg[s]))
            return cps

        def compute_chunk(ci, s):
            ba = bufa[s]
            bb = bufb[s]
            r0 = ci * ch_rows

            def group_body(g, carry2):
                gr = r0 + lax.shift_right_logical(g, 3)
                gl = lax.bitwise_and(g, 7) * 16
                w16 = wv[gr, pl.ds(gl, 16)]
                e0 = g * 16
                for k in range(16):
                    wvec = lax.gather(
                        w16, jnp.full((16, 1), k, jnp.int32),
                        lax.GatherDimensionNumbers(
                            offset_dims=(), collapsed_slice_dims=(0,),
                            start_index_map=(0,)),
                        slice_sizes=(1,),
                        mode=lax.GatherScatterMode.PROMISE_IN_BOUNDS)
                    e = e0 + k
                    lo = pl.ds(0, 16)
                    hi = pl.ds(16, 16)
                    ba[e, lo] = (ba[e, lo] + bb[e, lo]) * wvec
                    ba[e, hi] = (ba[e, hi] + bb[e, hi]) * wvec
                return carry2

            lax.fori_loop(0, ch // 16, group_body, 0)

        def fire_scatters(ci, s):
            r0 = ci * ch_rows
            cps = []
            for j in range(ch_rows):
                cps.append(pltpu.async_copy(
                    bufa[s].at[pl.ds(j * _LANE, _LANE)],
                    accum.at[dstv.at[r0 + j]], ss[s], add=True))
            return cps

        pend_scatter = [None, None]
        pend_gather = [None, None]
        pend_gather[0] = fire_gathers(0, 0)
        for ci in range(n_chunks):
            s = ci % 2
            o = 1 - s
            if ci + 1 < n_chunks:
                if pend_scatter[o] is not None:
                    for cp in pend_scatter[o]:
                        cp.wait()
                    pend_scatter[o] = None
                pend_gather[o] = fire_gathers(ci + 1, o)
            for cp in pend_gather[s]:
                cp.wait()
            compute_chunk(ci, s)
            pend_scatter[s] = fire_scatters(ci, s)
        for s in range(2):
            if pend_scatter[s] is not None:
                for cp in pend_scatter[s]:
                    cp.wait()

        plsc.subcore_barrier()
        pltpu.sync_copy(accum.at[pl.ds(base, ch)], bufa0.at[pl.ds(0, ch)])
        pltpu.sync_copy(bufa0.at[pl.ds(0, ch)],
                        out_h.at[cid, pl.ds(base, ch)])
        pltpu.sync_copy(accum.at[pl.ds(base + ch, rem)],
                        bufa1.at[pl.ds(0, rem)])
        pltpu.sync_copy(bufa1.at[pl.ds(0, rem)],
                        out_h.at[cid, pl.ds(base + ch, rem)])

    return sck(gia, gib, dstr, wr, ta, tb)


def kernel(x, edge_index, edge_type, edge_time, lambda_sym, beta, Wf, bf,
           Wr1, br1, Wr2, br2, W0, b0, W1, b1, W2, b2):
    N, in_dim = x.shape
    H = Wf.shape[1]
    R = Wr1.shape[0]
    E = edge_index.shape[1]
    out_dim = W0.shape[1]
    f32 = jnp.float32

    nb = 10
    bn = N // nb

    h0 = pl.pallas_call(
        _emb_body,
        grid=(nb,),
        in_specs=[
            pl.BlockSpec((bn, in_dim), lambda i: (i, 0)),
            pl.BlockSpec((in_dim, H), lambda i: (0, 0)),
            pl.BlockSpec((1, H), lambda i: (0, 0)),
        ],
        out_specs=pl.BlockSpec((bn, H), lambda i: (i, 0)),
        out_shape=jax.ShapeDtypeStruct((N, H), f32),
    )(x, Wf, bf.reshape(1, H))

    group = _NC * _NS * _LANE * 8
    e_pad = ((E + group - 1) // group) * group
    n_pad = ((N + 8 * _NS - 1) // (8 * _NS)) * (8 * _NS)
    erows = E // _LANE
    rows_pad = e_pad // _LANE
    tdim = edge_time.shape[1]
    ei3 = edge_index.astype(jnp.int32).reshape(2, erows, _LANE)
    ety2 = edge_type.astype(jnp.int32).reshape(erows, _LANE)
    ett3 = jnp.transpose(edge_time).reshape(tdim, erows, _LANE)
    gia_p, gib_p, dst_p, w_p = pl.pallas_call(
        functools.partial(_prep_body, r=R),
        grid=(1,),
        in_specs=[
            pl.BlockSpec((2, erows, _LANE), lambda i: (0, 0, 0)),
            pl.BlockSpec((erows, _LANE), lambda i: (0, 0)),
            pl.BlockSpec((tdim, erows, _LANE), lambda i: (0, 0, 0)),
            pl.BlockSpec((1, tdim), lambda i: (0, 0)),
            pl.BlockSpec((1, 1), lambda i: (0, 0)),
        ],
        out_specs=[
            pl.BlockSpec((rows_pad, _LANE), lambda i: (0, 0)),
            pl.BlockSpec((rows_pad, _LANE), lambda i: (0, 0)),
            pl.BlockSpec((rows_pad, _LANE), lambda i: (0, 0)),
            pl.BlockSpec((rows_pad, _LANE), lambda i: (0, 0)),
        ],
        out_shape=[
            jax.ShapeDtypeStruct((rows_pad, _LANE), jnp.int32),
            jax.ShapeDtypeStruct((rows_pad, _LANE), jnp.int32),
            jax.ShapeDtypeStruct((rows_pad, _LANE), jnp.int32),
            jax.ShapeDtypeStruct((rows_pad, _LANE), f32),
        ],
    )(ei3, ety2, ett3, beta.reshape(1, tdim), lambda_sym)

    rh = R * H
    wcat1t = jnp.transpose(Wr1[:, :H, :], (1, 0, 2)).reshape(H, rh)
    wcat1b = jnp.transpose(Wr1[:, H:, :], (1, 0, 2)).reshape(H, rh)
    wcat2t = jnp.transpose(Wr2[:, :H, :], (1, 0, 2)).reshape(H, rh)
    wcat2b = jnp.transpose(Wr2[:, H:, :], (1, 0, 2)).reshape(H, rh)
    wcat_spec = pl.BlockSpec((H, rh), lambda i: (0, 0))
    bcat_spec = pl.BlockSpec((1, rh), lambda i: (0, 0))
    t_out = [
        pl.BlockSpec((N, rh), lambda i: (0, 0)),
        pl.BlockSpec((N, rh), lambda i: (0, 0)),
    ]
    t_shape = [
        jax.ShapeDtypeStruct((N, rh), f32),
        jax.ShapeDtypeStruct((N, rh), f32),
    ]

    ta1, tb1 = pl.pallas_call(
        _tables_body,
        grid=(1,),
        in_specs=[
            pl.BlockSpec((N, H), lambda i: (0, 0)),
            wcat_spec, wcat_spec, bcat_spec,
        ],
        out_specs=t_out,
        out_shape=t_shape,
    )(h0, wcat1t, wcat1b, br1.reshape(1, rh))

    p1 = _sc_layer(ta1.reshape(R * N, H), tb1.reshape(R * N, H),
                   gia_p, gib_p, dst_p, w_p, n_pad, H)

    ta2, tb2, h1 = pl.pallas_call(
        _tables_sum_body,
        grid=(1,),
        in_specs=[
            pl.BlockSpec((_NC, n_pad, H), lambda i: (0, 0, 0)),
            wcat_spec, wcat_spec, bcat_spec,
        ],
        out_specs=t_out + [pl.BlockSpec((N, H), lambda i: (0, 0))],
        out_shape=t_shape + [jax.ShapeDtypeStruct((N, H), f32)],
    )(p1, wcat2t, wcat2b, br2.reshape(1, rh))

    p2 = _sc_layer(ta2.reshape(R * N, H), tb2.reshape(R * N, H),
                   gia_p, gib_p, dst_p, w_p, n_pad, H)

    out = pl.pallas_call(
        _final_body,
        grid=(nb,),
        in_specs=[
            pl.BlockSpec((bn, H), lambda i: (i, 0)),
            pl.BlockSpec((bn, H), lambda i: (i, 0)),
            pl.BlockSpec((_NC, bn, H), lambda i: (0, i, 0)),
            pl.BlockSpec((H, out_dim), lambda i: (0, 0)),
            pl.BlockSpec((1, out_dim), lambda i: (0, 0)),
            pl.BlockSpec((H, out_dim), lambda i: (0, 0)),
            pl.BlockSpec((1, out_dim), lambda i: (0, 0)),
            pl.BlockSpec((H, out_dim), lambda i: (0, 0)),
            pl.BlockSpec((1, out_dim), lambda i: (0, 0)),
        ],
        out_specs=pl.BlockSpec((bn, out_dim), lambda i: (i, 0)),
        out_shape=jax.ShapeDtypeStruct((N, out_dim), f32),
    )(h0, h1, p2, W0, b0.reshape(1, out_dim), W1, b1.reshape(1, out_dim),
      W2, b2.reshape(1, out_dim))

    return out
